# Initial kernel scaffold; baseline (speedup 1.0000x reference)
import jax
import jax.numpy as jnp
from jax.experimental import pallas as pl


def kernel(x, batch):
    raise NotImplementedError("write your pallas kernel here")


# speedup vs baseline: 53.5849x; 53.5849x over previous
import jax
import jax.numpy as jnp
from jax.experimental import pallas as pl

_NUM_SEGMENTS = 16
_BISECT_ITERS = 28
_NEG = -1e30


def _sparsemax_kernel(x_ref, b_ref, o_ref):
    x = x_ref[:, :]
    b = b_ref[:, :]

    ys = []
    maxs = []
    for s in range(_NUM_SEGMENTS):
        mask = b == s
        xs = jnp.where(mask, x, _NEG)
        m = jnp.max(xs)
        maxs.append(m)
        ys.append(jnp.where(mask, x - m, _NEG))

    def body(_, carry):
        lo, hi = carry
        new_lo = []
        new_hi = []
        for s in range(_NUM_SEGMENTS):
            mid = 0.5 * (lo[s] + hi[s])
            f = jnp.sum(jnp.maximum(ys[s] - mid, 0.0))
            pred = f >= 1.0
            new_lo.append(jnp.where(pred, mid, lo[s]))
            new_hi.append(jnp.where(pred, hi[s], mid))
        return tuple(new_lo), tuple(new_hi)

    lo0 = tuple(jnp.float32(-1.0) for _ in range(_NUM_SEGMENTS))
    hi0 = tuple(jnp.float32(0.0) for _ in range(_NUM_SEGMENTS))
    lo, hi = jax.lax.fori_loop(0, _BISECT_ITERS, body, (lo0, hi0))

    out = jnp.zeros_like(x)
    for s in range(_NUM_SEGMENTS):
        tau = 0.5 * (lo[s] + hi[s])
        out = jnp.where(b == s, jnp.maximum(x - maxs[s] - tau, 0.0), out)
    o_ref[:, :] = out


def kernel(x, batch):
    n = x.shape[0]
    rows = n // 128
    x2 = x.reshape(rows, 128)
    b2 = batch.reshape(rows, 128)
    out = pl.pallas_call(
        _sparsemax_kernel,
        out_shape=jax.ShapeDtypeStruct((rows, 128), jnp.float32),
    )(x2, b2)
    return out.reshape(n)
